# baseline (speedup 1.0000x reference)
import jax
import jax.numpy as jnp
from jax import lax
from jax.experimental import pallas as pl
from jax.experimental.pallas import tpu as pltpu
from jax.experimental.pallas import tpu_sc as plsc

B = 16384
L = 20
EMB = 32
TEXT_V = 10000
NC = 2
NS = 16
NW = NC * NS
BPW = B // NW
C = 32
NCHUNK = BPW // C
NGROUP = BPW // 16
VPT = TEXT_V // NS

_IOTA16 = None


def _body(title_hbm, tok_hbm, item_hbm, text_hbm, out_hbm,
          ids_v, id_rows, tok_v, tokT_c, rows_v, stage_v, inv_v, n0_v,
          out_c, t0_v, text_sh, sem, sem2):
    cid = lax.axis_index("c")
    sid = lax.axis_index("s")
    wid = sid * NC + cid
    base = wid * BPW
    iota = lax.iota(jnp.int32, 16)

    pltpu.sync_copy(title_hbm.at[pl.ds(base, BPW)], ids_v)
    pltpu.sync_copy(tok_hbm.at[pl.ds(base, BPW), :], tok_v)

    b1 = pltpu.async_copy(item_hbm.at[ids_v], id_rows, sem2)

    vbase = sid * VPT
    pltpu.sync_copy(text_hbm.at[pl.ds(vbase, VPT)], stage_v)
    pltpu.sync_copy(stage_v, text_sh.at[pl.ds(vbase, VPT)])

    pltpu.sync_copy(text_hbm.at[pl.ds(0, 1)], t0_v)

    @pl.loop(0, NGROUP)
    def _count(g):
        off = g * 16
        rows16 = off + iota
        n0i = jnp.zeros((16,), jnp.int32)
        for l in range(L):
            t = plsc.load_gather(tok_v, [rows16, jnp.full((16,), l, jnp.int32)])
            n0i = n0i + jnp.where(t == 0, 1, 0)
        n0f = n0i.astype(jnp.float32)
        cnt = jnp.float32(L) - n0f
        inv = jnp.float32(1.0) / jnp.maximum(cnt, jnp.float32(1.0))
        n0_v[pl.ds(off, 16)] = n0f
        inv_v[pl.ds(off, 16)] = inv

    plsc.subcore_barrier()

    @pl.loop(0, NCHUNK)
    def _chunk(c):
        cbase = c * C
        for l in range(L):
            lfull = jnp.full((16,), l, jnp.int32)
            for u in range(C // 16):
                rows16 = cbase + u * 16 + iota
                tokT_c[l, pl.ds(u * 16, 16)] = plsc.load_gather(
                    tok_v, [rows16, lfull])
        copies = []
        for l in range(L):
            copies.append(pltpu.async_copy(
                text_sh.at[tokT_c.at[l]],
                rows_v.at[l], sem))
        for cp in copies:
            cp.wait()
        for r in range(C):
            bl = cbase + r
            lane_b = jnp.full((16,), bl, jnp.int32)
            n0b = plsc.load_gather(n0_v, [lane_b])
            invb = plsc.load_gather(inv_v, [lane_b])
            for h in range(2):
                s = rows_v[0, r, pl.ds(h * 16, 16)]
                for l in range(1, L):
                    s = s + rows_v[l, r, pl.ds(h * 16, 16)]
                t0h = t0_v[0, pl.ds(h * 16, 16)]
                out_c[r, pl.ds(h * 16, 16)] = (s - n0b * t0h) * invb
        pltpu.sync_copy(out_c,
                        out_hbm.at[pl.ds(base + cbase, C), pl.ds(EMB, EMB)])

    b1.wait()
    pltpu.sync_copy(id_rows, out_hbm.at[pl.ds(base, BPW), pl.ds(0, EMB)])


_mesh = plsc.VectorSubcoreMesh(core_axis_name="c", subcore_axis_name="s")

_sc_call = pl.kernel(
    _body,
    out_type=jax.ShapeDtypeStruct((B, 2 * EMB), jnp.float32),
    mesh=_mesh,
    compiler_params=pltpu.CompilerParams(use_tc_tiling_on_sc=False,
                                         needs_layout_passes=False),
    scratch_types=[
        pltpu.VMEM((BPW,), jnp.int32),
        pltpu.VMEM((BPW, EMB), jnp.float32),
        pltpu.VMEM((BPW, L), jnp.int32),
        pltpu.VMEM((L, C), jnp.int32),
        pltpu.VMEM((L, C, EMB), jnp.float32),
        pltpu.VMEM((VPT, EMB), jnp.float32),
        pltpu.VMEM((BPW,), jnp.float32),
        pltpu.VMEM((BPW,), jnp.float32),
        pltpu.VMEM((C, EMB), jnp.float32),
        pltpu.VMEM((1, EMB), jnp.float32),
        pltpu.VMEM_SHARED((TEXT_V, EMB), jnp.float32),
        pltpu.SemaphoreType.DMA,
        pltpu.SemaphoreType.DMA,
    ],
)


def kernel(title_ids, title_token_ids, item_table, text_table):
    return _sc_call(title_ids, title_token_ids, item_table, text_table)
